# baseline (speedup 1.0000x reference)
import functools

import jax
import jax.numpy as jnp
from jax import lax
from jax.experimental import pallas as pl
from jax.experimental.pallas import tpu as pltpu
from jax.experimental.pallas import tpu_sc as plsc

N_EXP = 16
TOPK = 2
EPS = 0.05
ITERS = 20
BM = 1024


def _router_kernel(x_ref, w_ref, pit_ref, loss_ref, k_ref, b_ref):
    t = pl.program_id(0)
    nt = pl.num_programs(0)
    xb = x_ref[...]
    w = w_ref[...]
    lg = jax.lax.dot_general(
        w, xb, (((1,), (1,)), ((), ())), preferred_element_type=jnp.float32
    )
    la = lg * (-1.0 / EPS)
    m = jnp.max(la, axis=0, keepdims=True)
    kt = jnp.exp(la - m)
    k_ref[:, pl.ds(t * BM, BM)] = kt
    b_ref[:, pl.ds(t * BM, BM)] = 1.0 / jnp.sum(kt, axis=0, keepdims=True)

    @pl.when(t == nt - 1)
    def _finalize():
        scale = float(pit_ref.shape[1]) / N_EXP

        b = b_ref[...]
        for _ in range(ITERS - 1):
            k = k_ref[...]
            a = scale / jnp.sum(k * b, axis=1, keepdims=True)
            b = 1.0 / jnp.sum(k * a, axis=0, keepdims=True)
        k = k_ref[...]
        a = 1.0 / jnp.sum(k * b, axis=1, keepdims=True)
        pit = k * a * b
        pit_ref[...] = pit

        u = 1.0 / N_EXP
        load = jnp.mean(pit, axis=1, keepdims=True)
        loss_ref[...] = jnp.sum(
            u * (jnp.log(u) - jnp.log(load)), axis=(0, 1), keepdims=True
        )


def _make_sc_topk(n):
    info = plsc.get_sparse_core_info()
    nw = info.num_cores * info.num_subcores
    lanes = info.num_lanes
    per_w = n // nw
    groups = per_w // lanes
    mesh = plsc.VectorSubcoreMesh(core_axis_name="c", subcore_axis_name="s")

    @functools.partial(
        pl.kernel,
        mesh=mesh,
        out_type=jax.ShapeDtypeStruct((TOPK, n), jnp.int32),
        scratch_types=[
            pltpu.VMEM((N_EXP, per_w), jnp.float32),
            pltpu.VMEM((TOPK, per_w), jnp.int32),
        ],
    )
    def sc_topk(pit_hbm, idx_hbm, p_v, idx_v):
        wid = lax.axis_index("s") * info.num_cores + lax.axis_index("c")
        base = wid * per_w
        pltpu.sync_copy(pit_hbm.at[:, pl.ds(base, per_w)], p_v)

        def group(g, carry):
            off = g * lanes
            rows = [p_v[e, pl.ds(off, lanes)] for e in range(N_EXP)]
            m1 = rows[0]
            for e in range(1, N_EXP):
                m1 = jnp.maximum(m1, rows[e])
            i1 = jnp.full((lanes,), N_EXP, jnp.int32)
            for e in range(N_EXP - 1, -1, -1):
                i1 = jnp.where(rows[e] == m1, e, i1)
            neg = jnp.full((lanes,), -jnp.inf, jnp.float32)
            m2 = neg
            for e in range(N_EXP):
                m2 = jnp.maximum(m2, jnp.where(i1 == e, neg, rows[e]))
            i2 = jnp.full((lanes,), N_EXP, jnp.int32)
            for e in range(N_EXP - 1, -1, -1):
                i2 = jnp.where((rows[e] == m2) & (i1 != e), e, i2)
            idx_v[0, pl.ds(off, lanes)] = i1
            idx_v[1, pl.ds(off, lanes)] = i2
            return carry

        lax.fori_loop(0, groups, group, 0)
        pltpu.sync_copy(idx_v, idx_hbm.at[:, pl.ds(base, per_w)])

    return sc_topk


def kernel(x, gate_w, centroids):
    b, t, d = x.shape
    n = b * t
    x2 = x.reshape(n, d)
    pit, loss = pl.pallas_call(
        _router_kernel,
        grid=(n // BM,),
        in_specs=[
            pl.BlockSpec((BM, d), lambda i: (i, 0)),
            pl.BlockSpec((N_EXP, d), lambda i: (0, 0)),
        ],
        out_specs=[
            pl.BlockSpec((N_EXP, n), lambda i: (0, 0)),
            pl.BlockSpec((1, 1), lambda i: (0, 0)),
        ],
        out_shape=[
            jax.ShapeDtypeStruct((N_EXP, n), jnp.float32),
            jax.ShapeDtypeStruct((1, 1), jnp.float32),
        ],
        scratch_shapes=[
            pltpu.VMEM((N_EXP, n), jnp.float32),
            pltpu.VMEM((1, n), jnp.float32),
        ],
    )(x2, gate_w)
    idxt = _make_sc_topk(n)(pit)
    dispatch = pit.T.reshape(b, t, N_EXP)
    indices = idxt.T.reshape(b, t, TOPK)
    load_loss = loss[0, 0]
    return dispatch, indices, load_loss


# speedup vs baseline: 1.5525x; 1.5525x over previous
import jax
import jax.numpy as jnp
from jax.experimental import pallas as pl
from jax.experimental.pallas import tpu as pltpu

N_EXP = 16
TOPK = 2
EPS = 0.05
ITERS = 20
BM = 1024


def _router_kernel(x_ref, w_ref, pit_ref, idx_ref, loss_ref, k_ref, b_ref):
    t = pl.program_id(0)
    nt = pl.num_programs(0)
    xb = x_ref[...]
    w = w_ref[...]
    lg = jax.lax.dot_general(
        w, xb, (((1,), (1,)), ((), ())), preferred_element_type=jnp.float32
    )
    la = lg * (-1.0 / EPS)
    m = jnp.max(la, axis=0, keepdims=True)
    kt = jnp.exp(la - m)
    k_ref[:, pl.ds(t * BM, BM)] = kt
    b_ref[:, pl.ds(t * BM, BM)] = 1.0 / jnp.sum(kt, axis=0, keepdims=True)

    @pl.when(t == nt - 1)
    def _finalize():
        scale = float(pit_ref.shape[1]) / N_EXP

        b = b_ref[...]
        for _ in range(ITERS - 1):
            k = k_ref[...]
            a = scale / jnp.sum(k * b, axis=1, keepdims=True)
            b = 1.0 / jnp.sum(k * a, axis=0, keepdims=True)
        k = k_ref[...]
        a = 1.0 / jnp.sum(k * b, axis=1, keepdims=True)
        pit = k * a * b
        pit_ref[...] = pit

        iota = jax.lax.broadcasted_iota(jnp.int32, pit.shape, 0)
        mx1 = jnp.max(pit, axis=0, keepdims=True)
        i1 = jnp.min(jnp.where(pit == mx1, iota, N_EXP), axis=0, keepdims=True)
        masked = jnp.where(iota == i1, -jnp.inf, pit)
        mx2 = jnp.max(masked, axis=0, keepdims=True)
        i2 = jnp.min(jnp.where(masked == mx2, iota, N_EXP), axis=0, keepdims=True)
        idx_ref[0:1, :] = i1
        idx_ref[1:2, :] = i2

        u = 1.0 / N_EXP
        load = jnp.mean(pit, axis=1, keepdims=True)
        loss_ref[...] = jnp.sum(
            u * (jnp.log(u) - jnp.log(load)), axis=(0, 1), keepdims=True
        )


def kernel(x, gate_w, centroids):
    b, t, d = x.shape
    n = b * t
    x2 = x.reshape(n, d)
    pit, idxt, loss = pl.pallas_call(
        _router_kernel,
        grid=(n // BM,),
        in_specs=[
            pl.BlockSpec((BM, d), lambda i: (i, 0)),
            pl.BlockSpec((N_EXP, d), lambda i: (0, 0)),
        ],
        out_specs=[
            pl.BlockSpec((N_EXP, n), lambda i: (0, 0)),
            pl.BlockSpec((TOPK, n), lambda i: (0, 0)),
            pl.BlockSpec((1, 1), lambda i: (0, 0)),
        ],
        out_shape=[
            jax.ShapeDtypeStruct((N_EXP, n), jnp.float32),
            jax.ShapeDtypeStruct((TOPK, n), jnp.int32),
            jax.ShapeDtypeStruct((1, 1), jnp.float32),
        ],
        scratch_shapes=[
            pltpu.VMEM((N_EXP, n), jnp.float32),
            pltpu.VMEM((1, n), jnp.float32),
        ],
    )(x2, gate_w)
    dispatch = pit.T.reshape(b, t, N_EXP)
    indices = idxt.T.reshape(b, t, TOPK)
    load_loss = loss[0, 0]
    return dispatch, indices, load_loss
